# baseline (speedup 1.0000x reference)
import functools

import jax
import jax.numpy as jnp
from jax import lax
from jax.experimental import pallas as pl
from jax.experimental.pallas import tpu as pltpu
from jax.experimental.pallas import tpu_sc as plsc

L = 16

_GD = lax.GatherDimensionNumbers(
    offset_dims=(), collapsed_slice_dims=(0,), start_index_map=(0,))


def _lane_perm(v, ix2):
    return lax.gather(v, ix2, _GD, slice_sizes=(1,),
                      mode=lax.GatherScatterMode.PROMISE_IN_BOUNDS)


def _make_sc_edge_kernel(N, E, D, interpret=False):
    NC, NS = 2, 16
    NW = NC * NS
    KV = D // L
    DP = D + 8
    W = D + L
    B = 80
    EPW = E // NW
    assert EPW * NW == E and EPW % B == 0
    NCHUNK = EPW // B
    R1 = 640
    RLAST = N - (NS - 1) * R1
    ZR = 80
    assert RLAST > 0 and R1 % ZR == 0 and RLAST % ZR == 0

    mesh = plsc.VectorSubcoreMesh(core_axis_name="c", subcore_axis_name="s",
                                  num_cores=NC, num_subcores=NS)

    @functools.partial(
        pl.kernel,
        out_type=jax.ShapeDtypeStruct((NC, N, W), jnp.float32),
        mesh=mesh,
        scratch_types=[
            pltpu.VMEM((NCHUNK, B), jnp.int32),
            pltpu.VMEM((NCHUNK, B), jnp.int32),
            pltpu.VMEM((B, DP), jnp.float32),
            pltpu.VMEM((B, DP), jnp.float32),
            pltpu.VMEM((B, W), jnp.float32),
            pltpu.VMEM((B, DP), jnp.float32),
            pltpu.VMEM((B, DP), jnp.float32),
            pltpu.VMEM((B, W), jnp.float32),
            pltpu.VMEM((D,), jnp.float32),
            pltpu.VMEM((ZR, W), jnp.float32),
            pltpu.VMEM_SHARED((N, W), jnp.float32),
            pltpu.SemaphoreType.DMA,
            pltpu.SemaphoreType.DMA,
            pltpu.SemaphoreType.DMA,
            pltpu.SemaphoreType.DMA,
            pltpu.SemaphoreType.DMA,
            pltpu.SemaphoreType.DMA,
        ],
        compiler_params=pltpu.CompilerParams(use_tc_tiling_on_sc=False,
                                             needs_layout_passes=False),
        interpret=interpret,
    )
    def k(xl_hbm, xr_hbm, src_hbm, dst_hbm, att_hbm, out_hbm,
          src_all, dst_all, xl0, xr0, w0, xl1, xr1, w1, att_v, zbuf, acc_sh,
          gl0, gr0, gl1, gr1, ss0, ss1):
        c = lax.axis_index("c")
        s = lax.axis_index("s")
        wid = s * NC + c

        zv = jnp.zeros((L,), jnp.float32)

        def zb_body(i, _):
            for j in range(W // L):
                zbuf[i, pl.ds(j * L, L)] = zv
            return 0

        lax.fori_loop(0, ZR, zb_body, 0)
        row0 = pl.multiple_of(s * R1, 8)

        @pl.when(s < NS - 1)
        def _():
            for j in range(R1 // ZR):
                pltpu.sync_copy(zbuf, acc_sh.at[pl.ds(row0 + j * ZR, ZR)])

        @pl.when(s == NS - 1)
        def _():
            for j in range(RLAST // ZR):
                pltpu.sync_copy(zbuf, acc_sh.at[pl.ds(row0 + j * ZR, ZR)])

        pltpu.sync_copy(src_hbm.at[wid], src_all)
        pltpu.sync_copy(dst_hbm.at[wid], dst_all)
        pltpu.sync_copy(att_hbm, att_v)
        plsc.subcore_barrier()

        lane0 = lax.iota(jnp.int32, 16) == 0
        lanes = lax.iota(jnp.int32, L)
        spl = [jnp.full((L, 1), u, jnp.int32) for u in range(L)]
        att_k = [att_v[pl.ds(j * L, L)] for j in range(KV)]

        bufs = [(xl0, xr0, w0, gl0, gr0, ss0), (xl1, xr1, w1, gl1, gr1, ss1)]

        def start_gather(i, p):
            xl_r, xr_r = bufs[p][0], bufs[p][1]
            pltpu.async_copy(xl_hbm.at[src_all.at[i]], xl_r, bufs[p][3])
            pltpu.async_copy(xr_hbm.at[dst_all.at[i]], xr_r, bufs[p][4])

        def wait_gather(i, p):
            xl_r, xr_r = bufs[p][0], bufs[p][1]
            pltpu.make_async_copy(xl_hbm.at[src_all.at[i]], xl_r, bufs[p][3]).wait()
            pltpu.make_async_copy(xr_hbm.at[dst_all.at[i]], xr_r, bufs[p][4]).wait()

        def wait_scatter(i, p):
            w_r = bufs[p][2]
            pltpu.make_async_copy(w_r, acc_sh.at[dst_all.at[i]], bufs[p][5]).wait()

        def compute_chunk(p):
            xl_r, xr_r, w_r = bufs[p][0], bufs[p][1], bufs[p][2]

            def group_body(g, _):
                base = g * L
                rows = base + lanes
                accs = [None] * 4
                for kk in range(D):
                    cols = jnp.full((L,), kk, jnp.int32)
                    a = plsc.load_gather(xl_r, [rows, cols])
                    r = plsc.load_gather(xr_r, [rows, cols])
                    sm = a + r
                    lr = jnp.maximum(sm, sm * 0.2)
                    ab = _lane_perm(att_k[kk // L], spl[kk % L])
                    term = lr * ab
                    q = kk % 4
                    accs[q] = term if accs[q] is None else accs[q] + term
                acc = (accs[0] + accs[1]) + (accs[2] + accs[3]) \
                    if D >= 4 else accs[0]
                tg = jnp.exp(acc)
                for u in range(L):
                    b = base + u
                    tb = _lane_perm(tg, spl[u])
                    for j in range(KV):
                        w_r[b, pl.ds(j * L, L)] = xl_r[b, pl.ds(j * L, L)] * tb
                    w_r[b, pl.ds(D, L)] = jnp.where(lane0, tb, 0.0)
                return 0

            lax.fori_loop(0, B // L, group_body, 0)

        def start_scatter(i, p):
            pltpu.async_copy(bufs[p][2], acc_sh.at[dst_all.at[i]], bufs[p][5],
                             add=True)

        start_gather(0, 0)

        def pipe_body(jj, _):
            a = 2 * jj
            wait_gather(a, 0)
            start_gather(a + 1, 1)

            @pl.when(jj > 0)
            def _():
                wait_scatter(a - 2, 0)

            compute_chunk(0)
            start_scatter(a, 0)

            wait_gather(a + 1, 1)

            @pl.when(a + 2 < NCHUNK)
            def _():
                start_gather(a + 2, 0)

            @pl.when(jj > 0)
            def _():
                wait_scatter(a - 1, 1)

            compute_chunk(1)
            start_scatter(a + 1, 1)
            return 0

        lax.fori_loop(0, NCHUNK // 2, pipe_body, 0)

        if NCHUNK % 2:
            tl = NCHUNK - 1
            wait_gather(tl, 0)
            wait_scatter(tl - 2, 0)
            compute_chunk(0)
            start_scatter(tl, 0)
            wait_scatter(tl, 0)
            wait_scatter(tl - 1, 1)
        else:
            wait_scatter(NCHUNK - 2, 0)
            wait_scatter(NCHUNK - 1, 1)
        plsc.subcore_barrier()

        @pl.when(s < NS - 1)
        def _():
            pltpu.sync_copy(acc_sh.at[pl.ds(row0, R1)],
                            out_hbm.at[c, pl.ds(row0, R1)])

        @pl.when(s == NS - 1)
        def _():
            pltpu.sync_copy(acc_sh.at[pl.ds(row0, RLAST)],
                            out_hbm.at[c, pl.ds(row0, RLAST)])

    def call(xl, xr, src, dst, att):
        return k(xl, xr, src.reshape(NW, NCHUNK, B),
                 dst.reshape(NW, NCHUNK, B), att)

    return call


def _proj1_body(x_ref, wl_ref, wr_ref, wlin_ref, att_ref, bsum_ref,
                xl_ref, xr_ref, xlin_ref, selfw_ref, selft_ref):
    x = x_ref[...]
    xl = jnp.dot(x, wl_ref[...], preferred_element_type=jnp.float32)
    xr = jnp.dot(x, wr_ref[...], preferred_element_type=jnp.float32)
    xlin = jnp.dot(x, wlin_ref[...], preferred_element_type=jnp.float32)
    pad = jnp.zeros((xl.shape[0], 8), jnp.float32)
    xl_ref[...] = jnp.concatenate([xl, pad], axis=1)
    xr_ref[...] = jnp.concatenate([xr, pad], axis=1)
    xlin_ref[...] = xlin + bsum_ref[...]
    sm = xl + xr
    lr = jnp.maximum(sm, sm * 0.2)
    ts = jnp.exp(jnp.sum(lr * att_ref[...], axis=1, keepdims=True))
    selft_ref[...] = ts
    selfw_ref[...] = ts * xl


def _combine1_body(nm_ref, selfw_ref, selft_ref, xlin_ref,
                   wl2_ref, wr2_ref, wlin2_ref, att2_ref, bsum2_ref,
                   xl2_ref, xr2_ref, hlin_ref, selfw2_ref, selft2_ref):
    D = 64
    nm = nm_ref[...]
    numer = nm[0, :, :D] + nm[1, :, :D] + selfw_ref[...]
    denom = nm[0, :, D:D + 1] + nm[1, :, D:D + 1] + selft_ref[...]
    agg = numer / (denom + 1e-16)
    h = jnp.maximum(agg + xlin_ref[...], 0.0)
    xl2 = jnp.dot(h, wl2_ref[...], preferred_element_type=jnp.float32)
    xr2 = jnp.dot(h, wr2_ref[...], preferred_element_type=jnp.float32)
    hlin = jnp.dot(h, wlin2_ref[...], preferred_element_type=jnp.float32)
    pad = jnp.zeros((xl2.shape[0], 8), jnp.float32)
    xl2_ref[...] = jnp.concatenate([xl2, pad], axis=1)
    xr2_ref[...] = jnp.concatenate([xr2, pad], axis=1)
    hlin_ref[...] = hlin + bsum2_ref[...]
    sm = xl2 + xr2
    lr = jnp.maximum(sm, sm * 0.2)
    ts = jnp.exp(jnp.sum(lr * att2_ref[...], axis=1, keepdims=True))
    selft2_ref[...] = ts
    selfw2_ref[...] = ts * xl2


def _combine2_body(nm_ref, selfw2_ref, selft2_ref, hlin_ref, out_ref):
    D = 16
    nm = nm_ref[...]
    numer = nm[0, :, :D] + nm[1, :, :D] + selfw2_ref[...]
    denom = nm[0, :, D:D + 1] + nm[1, :, D:D + 1] + selft2_ref[...]
    o = numer / (denom + 1e-16) + hlin_ref[...]
    m = jnp.max(o, axis=1, keepdims=True)
    z = o - m
    lse = jnp.log(jnp.sum(jnp.exp(z), axis=1, keepdims=True))
    out_ref[...] = z - lse


def kernel(x, edge_index, Wl1, Wr1, att1, b1, Wlin1, blin1,
           Wl2, Wr2, att2, b2, Wlin2, blin2):
    N, F = x.shape
    E = edge_index.shape[1]
    HID = Wl1.shape[1]
    NCLS = Wl2.shape[1]
    BR = 1000
    G = N // BR

    src = edge_index[0]
    dst = edge_index[1]

    f32 = jnp.float32
    row_spec = lambda w: pl.BlockSpec((BR, w), lambda i: (i, 0))
    full_spec = lambda a, b: pl.BlockSpec((a, b), lambda i: (0, 0))

    xl1, xr1, xlin1, selfw1, selft1 = pl.pallas_call(
        _proj1_body,
        grid=(G,),
        in_specs=[row_spec(F), full_spec(F, HID), full_spec(F, HID),
                  full_spec(F, HID), full_spec(1, HID), full_spec(1, HID)],
        out_specs=[row_spec(HID + 8), row_spec(HID + 8), row_spec(HID),
                   row_spec(HID), row_spec(1)],
        out_shape=[jax.ShapeDtypeStruct((N, HID + 8), f32)] * 2
        + [jax.ShapeDtypeStruct((N, HID), f32)] * 2
        + [jax.ShapeDtypeStruct((N, 1), f32)],
    )(x, Wl1, Wr1, Wlin1, att1.reshape(1, HID),
      (b1 + blin1).reshape(1, HID))

    sc1 = _make_sc_edge_kernel(N, E, HID)
    nm1 = sc1(xl1, xr1, src, dst, att1)

    xl2, xr2, hlin2, selfw2, selft2 = pl.pallas_call(
        _combine1_body,
        grid=(G,),
        in_specs=[pl.BlockSpec((2, BR, HID + L), lambda i: (0, i, 0)),
                  row_spec(HID), row_spec(1), row_spec(HID),
                  full_spec(HID, NCLS), full_spec(HID, NCLS),
                  full_spec(HID, NCLS), full_spec(1, NCLS), full_spec(1, NCLS)],
        out_specs=[row_spec(NCLS + 8), row_spec(NCLS + 8), row_spec(NCLS),
                   row_spec(NCLS), row_spec(1)],
        out_shape=[jax.ShapeDtypeStruct((N, NCLS + 8), f32)] * 2
        + [jax.ShapeDtypeStruct((N, NCLS), f32)] * 2
        + [jax.ShapeDtypeStruct((N, 1), f32)],
    )(nm1, selfw1, selft1, xlin1, Wl2, Wr2, Wlin2,
      att2.reshape(1, NCLS), (b2 + blin2).reshape(1, NCLS))

    sc2 = _make_sc_edge_kernel(N, E, NCLS)
    nm2 = sc2(xl2, xr2, src, dst, att2)

    out = pl.pallas_call(
        _combine2_body,
        grid=(G,),
        in_specs=[pl.BlockSpec((2, BR, NCLS + L), lambda i: (0, i, 0)),
                  row_spec(NCLS), row_spec(1), row_spec(NCLS)],
        out_specs=row_spec(NCLS),
        out_shape=jax.ShapeDtypeStruct((N, NCLS), f32),
    )(nm2, selfw2, selft2, hlin2)

    return (out, edge_index)


# speedup vs baseline: 1.1109x; 1.1109x over previous
import functools

import jax
import jax.numpy as jnp
from jax import lax
from jax.experimental import pallas as pl
from jax.experimental.pallas import tpu as pltpu
from jax.experimental.pallas import tpu_sc as plsc

L = 16

_GD = lax.GatherDimensionNumbers(
    offset_dims=(), collapsed_slice_dims=(0,), start_index_map=(0,))


def _lane_perm(v, ix2):
    return lax.gather(v, ix2, _GD, slice_sizes=(1,),
                      mode=lax.GatherScatterMode.PROMISE_IN_BOUNDS)


def _make_sc_edge_kernel(N, E, D, packed=False):
    NC, NS = 2, 16
    NW = NC * NS
    KV = D // L
    if packed:
        PW = (D + 16) // 2
        CW = D // 2
        gdt = jnp.int32
        W = D + 8
    else:
        PW = D + 8
        gdt = jnp.float32
        W = D + L
    B = 80
    EPW = E // NW
    assert EPW * NW == E and EPW % B == 0
    NCHUNK = EPW // B
    R1 = 640
    RLAST = N - (NS - 1) * R1
    ZR = 40
    assert RLAST > 0 and R1 % ZR == 0 and RLAST % ZR == 0

    mesh = plsc.VectorSubcoreMesh(core_axis_name="c", subcore_axis_name="s",
                                  num_cores=NC, num_subcores=NS)

    @functools.partial(
        pl.kernel,
        out_type=jax.ShapeDtypeStruct((NC, N, W), jnp.float32),
        mesh=mesh,
        scratch_types=[
            pltpu.VMEM((NCHUNK, B), jnp.int32),
            pltpu.VMEM((NCHUNK, B), jnp.int32),
            pltpu.VMEM((B, PW), gdt),
            pltpu.VMEM((B, PW), gdt),
            pltpu.VMEM((B, W), jnp.float32),
            pltpu.VMEM((B, PW), gdt),
            pltpu.VMEM((B, PW), gdt),
            pltpu.VMEM((B, W), jnp.float32),
            pltpu.VMEM((D,), jnp.float32),
            pltpu.VMEM((ZR, W), jnp.float32),
            pltpu.VMEM_SHARED((N, W), jnp.float32),
            pltpu.SemaphoreType.DMA,
            pltpu.SemaphoreType.DMA,
            pltpu.SemaphoreType.DMA,
            pltpu.SemaphoreType.DMA,
            pltpu.SemaphoreType.DMA,
            pltpu.SemaphoreType.DMA,
        ],
        compiler_params=pltpu.CompilerParams(use_tc_tiling_on_sc=False,
                                             needs_layout_passes=False),
    )
    def k(xl_hbm, xr_hbm, src_hbm, dst_hbm, att_hbm, out_hbm,
          src_all, dst_all, xl0, xr0, w0, xl1, xr1, w1, att_v, zbuf, acc_sh,
          gl0, gr0, gl1, gr1, ss0, ss1):
        c = lax.axis_index("c")
        s = lax.axis_index("s")
        wid = s * NC + c

        zv = jnp.zeros((L,), jnp.float32)

        def zb_body(i, _):
            for j in range(W // L):
                zbuf[i, pl.ds(j * L, L)] = zv
            return 0

        lax.fori_loop(0, ZR, zb_body, 0)
        row0 = pl.multiple_of(s * R1, 8)

        @pl.when(s < NS - 1)
        def _():
            for j in range(R1 // ZR):
                pltpu.sync_copy(zbuf, acc_sh.at[pl.ds(row0 + j * ZR, ZR)])

        @pl.when(s == NS - 1)
        def _():
            for j in range(RLAST // ZR):
                pltpu.sync_copy(zbuf, acc_sh.at[pl.ds(row0 + j * ZR, ZR)])

        if packed:
            def wz_body(i, _):
                w0[i, pl.ds(W - L, L)] = zv
                w1[i, pl.ds(W - L, L)] = zv
                return 0

            lax.fori_loop(0, B, wz_body, 0)

        pltpu.sync_copy(src_hbm.at[wid], src_all)
        pltpu.sync_copy(dst_hbm.at[wid], dst_all)
        pltpu.sync_copy(att_hbm, att_v)
        plsc.subcore_barrier()

        lane0 = lax.iota(jnp.int32, 16) == 0
        lanes = lax.iota(jnp.int32, L)
        spl = [jnp.full((L, 1), u, jnp.int32) for u in range(L)]
        att_k = [att_v[pl.ds(j * L, L)] for j in range(KV)]

        bufs = [(xl0, xr0, w0, gl0, gr0, ss0), (xl1, xr1, w1, gl1, gr1, ss1)]

        def start_gather(i, p):
            xl_r, xr_r = bufs[p][0], bufs[p][1]
            pltpu.async_copy(xl_hbm.at[src_all.at[i]], xl_r, bufs[p][3])
            pltpu.async_copy(xr_hbm.at[dst_all.at[i]], xr_r, bufs[p][4])

        def wait_gather(i, p):
            xl_r, xr_r = bufs[p][0], bufs[p][1]
            pltpu.make_async_copy(xl_hbm.at[src_all.at[i]], xl_r, bufs[p][3]).wait()
            pltpu.make_async_copy(xr_hbm.at[dst_all.at[i]], xr_r, bufs[p][4]).wait()

        def wait_scatter(i, p):
            w_r = bufs[p][2]
            pltpu.make_async_copy(w_r, acc_sh.at[dst_all.at[i]], bufs[p][5]).wait()

        def unpack2(v32):
            return plsc.unpack(plsc.bitcast(v32, jnp.bfloat16),
                               format=plsc.PackFormat.INTERLEAVED,
                               preferred_element_type=jnp.float32)

        def compute_chunk(p):
            xl_r, xr_r, w_r = bufs[p][0], bufs[p][1], bufs[p][2]

            def group_body_packed(g, _):
                base = g * L
                rows = base + lanes
                accs = [None, None]
                for cc in range(CW):
                    cols = jnp.full((L,), cc, jnp.int32)
                    a2 = unpack2(plsc.load_gather(xl_r, [rows, cols]))
                    r2 = unpack2(plsc.load_gather(xr_r, [rows, cols]))
                    for h in range(2):
                        sm = a2[h] + r2[h]
                        lr = jnp.maximum(sm, sm * 0.2)
                        q = 2 * cc + h
                        ab = _lane_perm(att_k[q // L], spl[q % L])
                        term = lr * ab
                        accs[h] = term if accs[h] is None else accs[h] + term
                tg = jnp.exp(accs[0] + accs[1])
                for u in range(L):
                    b = base + u
                    tb = _lane_perm(tg, spl[u])
                    for hh in range(CW // L):
                        e, o = unpack2(xl_r[b, pl.ds(hh * L, L)])
                        w_r[b, pl.ds(2 * hh * L, L)] = e * tb
                        w_r[b, pl.ds((2 * hh + 1) * L, L)] = o * tb
                    plsc.store_scatter(
                        w_r, [jnp.full((L,), b, jnp.int32),
                              jnp.full((L,), D, jnp.int32)], tb, mask=lane0)
                return 0

            def group_body(g, _):
                base = g * L
                rows = base + lanes
                acc = None
                for kk in range(D):
                    cols = jnp.full((L,), kk, jnp.int32)
                    a = plsc.load_gather(xl_r, [rows, cols])
                    r = plsc.load_gather(xr_r, [rows, cols])
                    sm = a + r
                    lr = jnp.maximum(sm, sm * 0.2)
                    ab = _lane_perm(att_k[kk // L], spl[kk % L])
                    term = lr * ab
                    acc = term if acc is None else acc + term
                tg = jnp.exp(acc)
                for u in range(L):
                    b = base + u
                    tb = _lane_perm(tg, spl[u])
                    for j in range(KV):
                        w_r[b, pl.ds(j * L, L)] = xl_r[b, pl.ds(j * L, L)] * tb
                    w_r[b, pl.ds(D, L)] = jnp.where(lane0, tb, 0.0)
                return 0

            lax.fori_loop(0, B // L,
                          group_body_packed if packed else group_body, 0)

        def start_scatter(i, p):
            pltpu.async_copy(bufs[p][2], acc_sh.at[dst_all.at[i]], bufs[p][5],
                             add=True)

        start_gather(0, 0)

        def pipe_body(jj, _):
            a = 2 * jj
            wait_gather(a, 0)
            start_gather(a + 1, 1)

            @pl.when(jj > 0)
            def _():
                wait_scatter(a - 2, 0)

            compute_chunk(0)
            start_scatter(a, 0)

            wait_gather(a + 1, 1)

            @pl.when(a + 2 < NCHUNK)
            def _():
                start_gather(a + 2, 0)

            @pl.when(jj > 0)
            def _():
                wait_scatter(a - 1, 1)

            compute_chunk(1)
            start_scatter(a + 1, 1)
            return 0

        lax.fori_loop(0, NCHUNK // 2, pipe_body, 0)

        if NCHUNK % 2:
            tl = NCHUNK - 1
            wait_gather(tl, 0)
            wait_scatter(tl - 2, 0)
            compute_chunk(0)
            start_scatter(tl, 0)
            wait_scatter(tl, 0)
            wait_scatter(tl - 1, 1)
        else:
            wait_scatter(NCHUNK - 2, 0)
            wait_scatter(NCHUNK - 1, 1)
        plsc.subcore_barrier()

        @pl.when(s < NS - 1)
        def _():
            pltpu.sync_copy(acc_sh.at[pl.ds(row0, R1)],
                            out_hbm.at[c, pl.ds(row0, R1)])

        @pl.when(s == NS - 1)
        def _():
            pltpu.sync_copy(acc_sh.at[pl.ds(row0, RLAST)],
                            out_hbm.at[c, pl.ds(row0, RLAST)])

    def call(xl, xr, src, dst, att):
        return k(xl, xr, src.reshape(NW, NCHUNK, B),
                 dst.reshape(NW, NCHUNK, B), att)

    return call


def _proj1_body(x_ref, wl_ref, wr_ref, wlin_ref, att_ref, bsum_ref,
                xl_ref, xr_ref, xlin_ref, selfw_ref, selft_ref):
    x = x_ref[...]
    xl = jnp.dot(x, wl_ref[...], preferred_element_type=jnp.float32)
    xr = jnp.dot(x, wr_ref[...], preferred_element_type=jnp.float32)
    xlin = jnp.dot(x, wlin_ref[...], preferred_element_type=jnp.float32)
    xl_ref[...] = xl
    xr_ref[...] = xr
    xlin_ref[...] = xlin + bsum_ref[...]
    sm = xl + xr
    lr = jnp.maximum(sm, sm * 0.2)
    ts = jnp.exp(jnp.sum(lr * att_ref[...], axis=1, keepdims=True))
    selft_ref[...] = ts
    selfw_ref[...] = ts * xl


def _combine1_body(nm_ref, selfw_ref, selft_ref, xlin_ref,
                   wl2_ref, wr2_ref, wlin2_ref, att2_ref, bsum2_ref,
                   xl2_ref, xr2_ref, hlin_ref, selfw2_ref, selft2_ref):
    D = 64
    nm = nm_ref[...]
    numer = nm[0, :, :D] + nm[1, :, :D] + selfw_ref[...]
    denom = nm[0, :, D:D + 1] + nm[1, :, D:D + 1] + selft_ref[...]
    agg = numer / (denom + 1e-16)
    h = jnp.maximum(agg + xlin_ref[...], 0.0)
    xl2 = jnp.dot(h, wl2_ref[...], preferred_element_type=jnp.float32)
    xr2 = jnp.dot(h, wr2_ref[...], preferred_element_type=jnp.float32)
    hlin = jnp.dot(h, wlin2_ref[...], preferred_element_type=jnp.float32)
    pad = jnp.zeros((xl2.shape[0], 8), jnp.float32)
    xl2_ref[...] = jnp.concatenate([xl2, pad], axis=1)
    xr2_ref[...] = jnp.concatenate([xr2, pad], axis=1)
    hlin_ref[...] = hlin + bsum2_ref[...]
    sm = xl2 + xr2
    lr = jnp.maximum(sm, sm * 0.2)
    ts = jnp.exp(jnp.sum(lr * att2_ref[...], axis=1, keepdims=True))
    selft2_ref[...] = ts
    selfw2_ref[...] = ts * xl2


def _combine2_body(nm_ref, selfw2_ref, selft2_ref, hlin_ref, out_ref):
    D = 16
    nm = nm_ref[...]
    numer = nm[0, :, :D] + nm[1, :, :D] + selfw2_ref[...]
    denom = nm[0, :, D:D + 1] + nm[1, :, D:D + 1] + selft2_ref[...]
    o = numer / (denom + 1e-16) + hlin_ref[...]
    m = jnp.max(o, axis=1, keepdims=True)
    z = o - m
    lse = jnp.log(jnp.sum(jnp.exp(z), axis=1, keepdims=True))
    out_ref[...] = z - lse


def kernel(x, edge_index, Wl1, Wr1, att1, b1, Wlin1, blin1,
           Wl2, Wr2, att2, b2, Wlin2, blin2):
    N, F = x.shape
    E = edge_index.shape[1]
    HID = Wl1.shape[1]
    NCLS = Wl2.shape[1]
    BR = 1000
    G = N // BR

    src = edge_index[0]
    dst = edge_index[1]

    f32 = jnp.float32
    row_spec = lambda w: pl.BlockSpec((BR, w), lambda i: (i, 0))
    full_spec = lambda a, b: pl.BlockSpec((a, b), lambda i: (0, 0))

    xl1, xr1, xlin1, selfw1, selft1 = pl.pallas_call(
        _proj1_body,
        grid=(G,),
        in_specs=[row_spec(F), full_spec(F, HID), full_spec(F, HID),
                  full_spec(F, HID), full_spec(1, HID), full_spec(1, HID)],
        out_specs=[row_spec(HID), row_spec(HID), row_spec(HID),
                   row_spec(HID), row_spec(1)],
        out_shape=[jax.ShapeDtypeStruct((N, HID), f32)] * 4
        + [jax.ShapeDtypeStruct((N, 1), f32)],
    )(x, Wl1, Wr1, Wlin1, att1.reshape(1, HID),
      (b1 + blin1).reshape(1, HID))

    invp = jnp.array([32 * (q // 32) + 16 * (q % 2) + ((q % 32) // 2)
                      for q in range(HID)], jnp.int32)

    def pack_bf16(a):
        ap = a[:, invp].astype(jnp.bfloat16).reshape(N, HID // 2, 2)
        ai = jax.lax.bitcast_convert_type(ap, jnp.int32)
        return jnp.pad(ai, ((0, 0), (0, 8)))

    sc1 = _make_sc_edge_kernel(N, E, HID, packed=True)
    nm1 = sc1(pack_bf16(xl1), pack_bf16(xr1), src, dst, att1[invp])

    xl2, xr2, hlin2, selfw2, selft2 = pl.pallas_call(
        _combine1_body,
        grid=(G,),
        in_specs=[pl.BlockSpec((2, BR, HID + 8), lambda i: (0, i, 0)),
                  row_spec(HID), row_spec(1), row_spec(HID),
                  full_spec(HID, NCLS), full_spec(HID, NCLS),
                  full_spec(HID, NCLS), full_spec(1, NCLS), full_spec(1, NCLS)],
        out_specs=[row_spec(NCLS + 8), row_spec(NCLS + 8), row_spec(NCLS),
                   row_spec(NCLS), row_spec(1)],
        out_shape=[jax.ShapeDtypeStruct((N, NCLS + 8), f32)] * 2
        + [jax.ShapeDtypeStruct((N, NCLS), f32)] * 2
        + [jax.ShapeDtypeStruct((N, 1), f32)],
    )(nm1, selfw1, selft1, xlin1, Wl2, Wr2, Wlin2,
      att2.reshape(1, NCLS), (b2 + blin2).reshape(1, NCLS))

    sc2 = _make_sc_edge_kernel(N, E, NCLS)
    nm2 = sc2(xl2, xr2, src, dst, att2)

    out = pl.pallas_call(
        _combine2_body,
        grid=(G,),
        in_specs=[pl.BlockSpec((2, BR, NCLS + L), lambda i: (0, i, 0)),
                  row_spec(NCLS), row_spec(1), row_spec(NCLS)],
        out_specs=row_spec(NCLS),
        out_shape=jax.ShapeDtypeStruct((N, NCLS), f32),
    )(nm2, selfw2, selft2, hlin2)

    return (out, edge_index)
